# baseline (speedup 1.0000x reference)
import functools

import jax
import jax.numpy as jnp
from jax import lax
from jax.experimental import pallas as pl
from jax.experimental.pallas import tpu as pltpu
from jax.experimental.pallas import tpu_sc as plsc

_B, _N, _MAP, _DY, _DX = 4, 512, 4, 8, 6
_D = 256
_TE = 6
_TD = _TE * _DX
_K = _N * _MAP * _DY
_KT = 2048
_NBLK = _K // _KT
_NC, _NS = 2, 16
_ROWS = 64
_NCD = _D // 16


def _tc_t_body(x_ref, t2vw_ref, t2vb_ref, vtw_ref, vtb_ref, given_ref, t_ref):
    x = x_ref[0]
    xn = jnp.where(jnp.isnan(x), 0.0, x)
    xrep = jnp.repeat(xn, _TE, axis=1)
    xa = xrep * t2vw_ref[...] + t2vb_ref[...]
    col = lax.broadcasted_iota(jnp.int32, (_N, _TD), 1)
    tv = jnp.where(col % _TE == 0, xa, jnp.sin(xa))
    tt = jnp.dot(tv, vtw_ref[:_TD, :], preferred_element_type=jnp.float32)
    t_ref[0] = tt + vtb_ref[...] + given_ref[1:2, :]


def _asm_body(t_ref, y_ref, yg_ref, local_ref, wrow_ref, gdif_ref, val_ref):
    t_exp = jnp.tile(t_ref[0], (_KT // _N, 1))
    local_exp = jnp.repeat(local_ref[...], 32, axis=0)
    yv = y_ref[0, 0]
    yc = jnp.where(jnp.isnan(yv), 0.0, yv)
    gmask = jnp.isnan(yg_ref[0, 0])
    gcor = jnp.where(gmask, gdif_ref[...], 0.0)
    val_ref[0] = t_exp + local_exp + yc * wrow_ref[...] + gcor


def _sc_body(space_hbm, space_out, var_out, rowbuf, varbuf, sem):
    wid = lax.axis_index("s") * _NC + lax.axis_index("c")
    b = wid // _NBLK
    seg = lax.rem(wid, _NBLK)
    pltpu.sync_copy(space_hbm.at[pl.ds(seg, 1)], rowbuf.at[pl.ds(0, 1)])
    svec = [rowbuf[0, pl.ds(d * 16, 16)] for d in range(_NCD)]
    for r in range(1, _ROWS):
        for d in range(_NCD):
            rowbuf[r, pl.ds(d * 16, 16)] = svec[d]
    vv = jnp.full((16,), seg, jnp.int32)
    for q in range(_KT // 16):
        varbuf[pl.ds(q * 16, 16)] = vv
    base = seg * _KT
    cps = [pltpu.async_copy(rowbuf,
                            space_out.at[b, pl.ds(base + i * _ROWS, _ROWS)],
                            sem)
           for i in range(_KT // _ROWS)]
    cps.append(pltpu.async_copy(varbuf, var_out.at[b, pl.ds(base, _KT)], sem))
    for cp in cps:
        cp.wait()


def kernel(x, y, t2v_w, t2v_b, local_table, vt_w, vt_b, space_table,
           given_table):
    batch = x.shape[0]
    t2vw_f = t2v_w.reshape(1, _TD)
    t2vb_f = t2v_b.reshape(1, _TD)
    vtb_f = vt_b.reshape(1, _D)

    sc_fill = functools.partial(
        pl.kernel,
        out_type=[
            jax.ShapeDtypeStruct((batch, _K, _D), jnp.float32),
            jax.ShapeDtypeStruct((batch, _K), jnp.int32),
        ],
        mesh=plsc.VectorSubcoreMesh(core_axis_name="c", subcore_axis_name="s"),
        scratch_types=[
            pltpu.VMEM((_ROWS, _D), jnp.float32),
            pltpu.VMEM((_KT,), jnp.int32),
            pltpu.SemaphoreType.DMA,
        ],
    )(_sc_body)
    space_emb, var_idx = sc_fill(space_table)

    t_tab = pl.pallas_call(
        _tc_t_body,
        grid=(batch,),
        in_specs=[
            pl.BlockSpec((1, _N, _DX), lambda b: (b, 0, 0)),
            pl.BlockSpec((1, _TD), lambda b: (0, 0)),
            pl.BlockSpec((1, _TD), lambda b: (0, 0)),
            pl.BlockSpec((_TD + 1, _D), lambda b: (0, 0)),
            pl.BlockSpec((1, _D), lambda b: (0, 0)),
            pl.BlockSpec((2, _D), lambda b: (0, 0)),
        ],
        out_specs=pl.BlockSpec((1, _N, _D), lambda b: (b, 0, 0)),
        out_shape=jax.ShapeDtypeStruct((batch, _N, _D), jnp.float32),
    )(x, t2vw_f, t2vb_f, vt_w, vtb_f, given_table)

    y_flat = y.reshape(batch, _NBLK, _KT, 1)
    yg_flat = jnp.transpose(y, (0, 1, 3, 2)).reshape(batch, _NBLK, _KT, 1)
    wrow = vt_w[_TD:_TD + 1, :]
    gdif = given_table[0:1, :] - given_table[1:2, :]

    val = pl.pallas_call(
        _asm_body,
        grid=(batch, _NBLK),
        in_specs=[
            pl.BlockSpec((1, _N, _D), lambda b, c: (b, 0, 0)),
            pl.BlockSpec((1, 1, _KT, 1), lambda b, c: (b, c, 0, 0)),
            pl.BlockSpec((1, 1, _KT, 1), lambda b, c: (b, c, 0, 0)),
            pl.BlockSpec((_KT // 32, _D), lambda b, c: (c, 0)),
            pl.BlockSpec((1, _D), lambda b, c: (0, 0)),
            pl.BlockSpec((1, _D), lambda b, c: (0, 0)),
        ],
        out_specs=pl.BlockSpec((1, _KT, _D), lambda b, c: (b, c, 0)),
        out_shape=jax.ShapeDtypeStruct((batch, _K, _D), jnp.float32),
    )(t_tab, y_flat, yg_flat, local_table, wrow, gdif)
    return (val, space_emb, var_idx)


# speedup vs baseline: 1.1058x; 1.1058x over previous
import jax
import jax.numpy as jnp
from jax import lax
from jax.experimental import pallas as pl
from jax.experimental.pallas import tpu as pltpu

_B, _N, _MAP, _DY, _DX = 4, 512, 4, 8, 6
_D = 256
_TE = 6
_TD = _TE * _DX
_K = _N * _MAP * _DY
_KT = 2048
_NBLK = _K // _KT


def _body(x_ref, y_ref, yg_ref, t2vw_ref, t2vb_ref, local_ref, vtw_ref,
          vtb_ref, space_ref, given_ref, val_ref, space_out_ref, var_ref,
          t_scr):
    c = pl.program_id(1)

    @pl.when(c == 0)
    def _compute_t():
        x = x_ref[0]
        xn = jnp.where(jnp.isnan(x), 0.0, x)
        xrep = jnp.repeat(xn, _TE, axis=1)
        xa = xrep * t2vw_ref[...] + t2vb_ref[...]
        col = lax.broadcasted_iota(jnp.int32, (_N, _TD), 1)
        tv = jnp.where(col % _TE == 0, xa, jnp.sin(xa))
        tt = jnp.dot(tv, vtw_ref[:_TD, :], preferred_element_type=jnp.float32)
        t_scr[...] = tt + vtb_ref[...] + given_ref[1:2, :]

    t_exp = jnp.tile(t_scr[...], (_KT // _N, 1))
    local_exp = jnp.repeat(local_ref[...], 32, axis=0)
    yv = y_ref[0, 0]
    yc = jnp.where(jnp.isnan(yv), 0.0, yv)
    gmask = jnp.isnan(yg_ref[0, 0])
    gdif = given_ref[0:1, :] - given_ref[1:2, :]
    gcor = jnp.where(gmask, gdif, 0.0)
    wrow = vtw_ref[_TD:_TD + 1, :]
    val_ref[0] = t_exp + local_exp + yc * wrow + gcor
    rows = space_ref[...]
    rsel = lax.broadcasted_iota(jnp.int32, (_DY, 1), 0) == c
    srow = jnp.sum(jnp.where(rsel, rows, 0.0), axis=0, keepdims=True)
    space_out_ref[0] = jnp.broadcast_to(srow, (_KT, _D))
    var_ref[0, 0] = jnp.full((1, _KT), c, jnp.int32)


def kernel(x, y, t2v_w, t2v_b, local_table, vt_w, vt_b, space_table,
           given_table):
    batch = x.shape[0]
    y_flat = y.reshape(batch, _NBLK, _KT, 1)
    yg_flat = jnp.transpose(y, (0, 1, 3, 2)).reshape(batch, _NBLK, _KT, 1)
    t2vw_f = t2v_w.reshape(1, _TD)
    t2vb_f = t2v_b.reshape(1, _TD)
    vtb_f = vt_b.reshape(1, _D)

    val, space_emb, var4 = pl.pallas_call(
        _body,
        grid=(batch, _NBLK),
        in_specs=[
            pl.BlockSpec((1, _N, _DX), lambda b, c: (b, 0, 0)),
            pl.BlockSpec((1, 1, _KT, 1), lambda b, c: (b, c, 0, 0)),
            pl.BlockSpec((1, 1, _KT, 1), lambda b, c: (b, c, 0, 0)),
            pl.BlockSpec((1, _TD), lambda b, c: (0, 0)),
            pl.BlockSpec((1, _TD), lambda b, c: (0, 0)),
            pl.BlockSpec((_KT // 32, _D), lambda b, c: (c, 0)),
            pl.BlockSpec((_TD + 1, _D), lambda b, c: (0, 0)),
            pl.BlockSpec((1, _D), lambda b, c: (0, 0)),
            pl.BlockSpec((_DY, _D), lambda b, c: (0, 0)),
            pl.BlockSpec((2, _D), lambda b, c: (0, 0)),
        ],
        out_specs=[
            pl.BlockSpec((1, _KT, _D), lambda b, c: (b, c, 0)),
            pl.BlockSpec((1, _KT, _D), lambda b, c: (b, c, 0)),
            pl.BlockSpec((1, 1, 1, _KT), lambda b, c: (b, c, 0, 0)),
        ],
        out_shape=[
            jax.ShapeDtypeStruct((batch, _K, _D), jnp.float32),
            jax.ShapeDtypeStruct((batch, _K, _D), jnp.float32),
            jax.ShapeDtypeStruct((batch, _NBLK, 1, _KT), jnp.int32),
        ],
        scratch_shapes=[pltpu.VMEM((_N, _D), jnp.float32)],
    )(x, y_flat, yg_flat, t2vw_f, t2vb_f, local_table, vt_w, vtb_f,
      space_table, given_table)
    return (val, space_emb, var4.reshape(batch, _K))
